# baseline (speedup 1.0000x reference)
import functools

import jax
import jax.numpy as jnp
from jax import lax
from jax.experimental import pallas as pl
from jax.experimental.pallas import tpu as pltpu


def _silu(u):
    return u / (1.0 + jnp.exp(-u))


def _sigmoid(u):
    return 1.0 / (1.0 + jnp.exp(-u))


def _dot_nt(a, b):
    return lax.dot_general(a, b, (((1,), (1,)), ((), ())),
                           preferred_element_type=jnp.float32)


def _dot_nt_bf16(a, b):
    return lax.dot_general(a.astype(jnp.bfloat16), b.astype(jnp.bfloat16),
                           (((1,), (1,)), ((), ())),
                           preferred_element_type=jnp.float32)


def _router_body(x_ref, gw_ref, cw_ref):
    x = x_ref[...]
    logits = _dot_nt(x, gw_ref[...])
    T, E = logits.shape
    lane = lax.broadcasted_iota(jnp.int32, (T, E), 1)
    m1 = jnp.max(logits, axis=1, keepdims=True)
    i1 = jnp.min(jnp.where(logits == m1, lane, E), axis=1, keepdims=True)
    masked = jnp.where(lane == i1, -jnp.inf, logits)
    m2 = jnp.max(masked, axis=1, keepdims=True)
    i2 = jnp.min(jnp.where(masked == m2, lane, E), axis=1, keepdims=True)
    w1 = 1.0 / (1.0 + jnp.exp(m2 - m1))
    w2 = 1.0 - w1
    cw_ref[...] = jnp.where(lane == i1, w1, 0.0) + jnp.where(lane == i2, w2, 0.0)


def _router(x, gate_w):
    T, D = x.shape
    E = gate_w.shape[0]
    return pl.pallas_call(
        _router_body,
        out_shape=jax.ShapeDtypeStruct((T, E), jnp.float32),
    )(x, gate_w)


def _moe_body(x_ref, wg_ref, wu_ref, wd_ref, cw_ref, out_ref, acc_ref, *, tb_sz):
    e = pl.program_id(0)
    tb = pl.program_id(1)
    xb = x_ref[...]
    g = _dot_nt_bf16(xb, wg_ref[0])
    u = _dot_nt_bf16(xb, wu_ref[0])
    h = g * _silu(u)
    y = _dot_nt_bf16(h, wd_ref[0])
    E = cw_ref.shape[1]
    lane = lax.broadcasted_iota(jnp.int32, (tb_sz, E), 1)
    tokw = jnp.sum(cw_ref[...] * jnp.where(lane == e, 1.0, 0.0),
                   axis=1, keepdims=True)
    contrib = y * tokw
    sl = pl.ds(tb * tb_sz, tb_sz)

    @pl.when(e == 0)
    def _():
        acc_ref[sl, :] = contrib

    @pl.when(e > 0)
    def _():
        acc_ref[sl, :] = acc_ref[sl, :] + contrib

    out_ref[...] = acc_ref[sl, :]


def _moe(x, Wg, Wu, Wd, cw):
    T, D = x.shape
    E, F, _ = Wg.shape
    TB = min(256, T)
    nb = T // TB
    body = functools.partial(_moe_body, tb_sz=TB)
    return pl.pallas_call(
        body,
        grid=(E, nb),
        in_specs=[
            pl.BlockSpec((TB, D), lambda e, tb: (tb, 0)),
            pl.BlockSpec((1, F, D), lambda e, tb: (e, 0, 0)),
            pl.BlockSpec((1, F, D), lambda e, tb: (e, 0, 0)),
            pl.BlockSpec((1, D, F), lambda e, tb: (e, 0, 0)),
            pl.BlockSpec((TB, E), lambda e, tb: (tb, 0)),
        ],
        out_specs=pl.BlockSpec((TB, D), lambda e, tb: (tb, 0)),
        out_shape=jax.ShapeDtypeStruct((T, D), jnp.float32),
        scratch_shapes=[pltpu.VMEM((T, D), jnp.float32)],
    )(x, Wg, Wu, Wd, cw)


def _shared_body(x_ref, swg_ref, swu_ref, swd_ref, sg_ref, routed_ref, out_ref):
    xb = x_ref[...]
    g = _dot_nt_bf16(xb, swg_ref[...])
    u = _dot_nt_bf16(xb, swu_ref[...])
    h = g * _silu(u)
    se = _dot_nt_bf16(h, swd_ref[...])
    gate = _sigmoid(_dot_nt(xb, sg_ref[...]))
    out_ref[...] = routed_ref[...] + se * gate


def _shared(x, sWg, sWu, sWd, s_gate, routed):
    T, D = x.shape
    S = sWg.shape[0]
    TB = min(256, T)
    nb = T // TB
    return pl.pallas_call(
        _shared_body,
        grid=(nb,),
        in_specs=[
            pl.BlockSpec((TB, D), lambda tb: (tb, 0)),
            pl.BlockSpec((S, D), lambda tb: (0, 0)),
            pl.BlockSpec((S, D), lambda tb: (0, 0)),
            pl.BlockSpec((D, S), lambda tb: (0, 0)),
            pl.BlockSpec((1, D), lambda tb: (0, 0)),
            pl.BlockSpec((TB, D), lambda tb: (tb, 0)),
        ],
        out_specs=pl.BlockSpec((TB, D), lambda tb: (tb, 0)),
        out_shape=jax.ShapeDtypeStruct((T, D), jnp.float32),
    )(x, sWg, sWu, sWd, s_gate, routed)


def kernel(hidden_states, gate_w, Wg, Wu, Wd, sWg, sWu, sWd, s_gate):
    x = hidden_states
    cw = _router(x, gate_w)
    routed = _moe(x, Wg, Wu, Wd, cw)
    return _shared(x, sWg, sWu, sWd, s_gate, routed)


# speedup vs baseline: 1.3546x; 1.3546x over previous
import functools

import jax
import jax.numpy as jnp
from jax import lax
from jax.experimental import pallas as pl
from jax.experimental.pallas import tpu as pltpu


def _silu(u):
    return u / (1.0 + jnp.exp(-u))


def _sigmoid(u):
    return 1.0 / (1.0 + jnp.exp(-u))


def _dot_nt(a, b):
    return lax.dot_general(a, b, (((1,), (1,)), ((), ())),
                           preferred_element_type=jnp.float32)


def _dot_nt_bf16(a, b):
    return lax.dot_general(a.astype(jnp.bfloat16), b.astype(jnp.bfloat16),
                           (((1,), (1,)), ((), ())),
                           preferred_element_type=jnp.float32)


def _combined_weights(logits):
    T, E = logits.shape
    lane = lax.broadcasted_iota(jnp.int32, (T, E), 1)
    m1 = jnp.max(logits, axis=1, keepdims=True)
    i1 = jnp.min(jnp.where(logits == m1, lane, E), axis=1, keepdims=True)
    masked = jnp.where(lane == i1, -jnp.inf, logits)
    m2 = jnp.max(masked, axis=1, keepdims=True)
    i2 = jnp.min(jnp.where(masked == m2, lane, E), axis=1, keepdims=True)
    w1 = 1.0 / (1.0 + jnp.exp(m2 - m1))
    w2 = 1.0 - w1
    return jnp.where(lane == i1, w1, 0.0) + jnp.where(lane == i2, w2, 0.0)


def _fused_body(x_ref, gw_ref, wg_ref, wu_ref, wd_ref,
                swg_ref, swu_ref, swd_ref, sg_ref,
                out_ref, cw_ref, *, tbs, n_exp):
    e = pl.program_id(0)
    tb = pl.program_id(1)
    sl = pl.ds(tb * tbs, tbs)

    @pl.when((e == 0) & (tb == 0))
    def _():
        cw_ref[...] = _combined_weights(_dot_nt(x_ref[...], gw_ref[...]))

    xs = x_ref[sl, :]

    @pl.when(e < n_exp)
    def _():
        g = _dot_nt_bf16(xs, wg_ref[0])
        u = _dot_nt_bf16(xs, wu_ref[0])
        y = _dot_nt_bf16(g * _silu(u), wd_ref[0])
        lane = lax.broadcasted_iota(jnp.int32, (tbs, n_exp), 1)
        tokw = jnp.sum(cw_ref[sl, :] * jnp.where(lane == e, 1.0, 0.0),
                       axis=1, keepdims=True)
        contrib = y * tokw

        @pl.when(e == 0)
        def _():
            out_ref[sl, :] = contrib

        @pl.when(e > 0)
        def _():
            out_ref[sl, :] = out_ref[sl, :] + contrib

    @pl.when(e >= n_exp)
    def _():
        g = _dot_nt_bf16(xs, swg_ref[...])
        u = _dot_nt_bf16(xs, swu_ref[...])
        y = _dot_nt_bf16(g * _silu(u), swd_ref[...])
        gate = _sigmoid(_dot_nt(xs, sg_ref[...]))
        out_ref[sl, :] = out_ref[sl, :] + y * gate


def kernel(hidden_states, gate_w, Wg, Wu, Wd, sWg, sWu, sWd, s_gate):
    x = hidden_states
    T, D = x.shape
    E, F, _ = Wg.shape
    S = sWg.shape[0]
    n_sh = S // F
    TBS = min(1024, T)
    ntb = T // TBS
    body = functools.partial(_fused_body, tbs=TBS, n_exp=E)
    return pl.pallas_call(
        body,
        grid=(E + n_sh, ntb),
        in_specs=[
            pl.BlockSpec((T, D), lambda e, tb: (0, 0)),
            pl.BlockSpec((E, D), lambda e, tb: (0, 0)),
            pl.BlockSpec((1, F, D), lambda e, tb: (jnp.minimum(e, E - 1), 0, 0)),
            pl.BlockSpec((1, F, D), lambda e, tb: (jnp.minimum(e, E - 1), 0, 0)),
            pl.BlockSpec((1, D, F), lambda e, tb: (jnp.minimum(e, E - 1), 0, 0)),
            pl.BlockSpec((F, D), lambda e, tb: (jnp.maximum(e - E, 0), 0)),
            pl.BlockSpec((F, D), lambda e, tb: (jnp.maximum(e - E, 0), 0)),
            pl.BlockSpec((D, F), lambda e, tb: (0, jnp.maximum(e - E, 0))),
            pl.BlockSpec((1, D), lambda e, tb: (0, 0)),
        ],
        out_specs=pl.BlockSpec((T, D), lambda e, tb: (0, 0)),
        out_shape=jax.ShapeDtypeStruct((T, D), jnp.float32),
        scratch_shapes=[pltpu.VMEM((T, E), jnp.float32)],
    )(x, gate_w, Wg, Wu, Wd, sWg, sWu, sWd, s_gate)
